# Initial kernel scaffold; baseline (speedup 1.0000x reference)
import functools

import jax
import jax.numpy as jnp
from jax import lax
from jax.experimental import pallas as pl
from jax.experimental.pallas import tpu as pltpu
from jax.experimental.pallas import tpu_sc as plsc

N = 1000
NP = 1024
NCH = NP // 16
G = 50
GP = 64
GT2 = G * G
GT2P = 2512
NGCH = GT2P // 16

NMS_THR = 0.5
EPS = 1e-6
LN2 = 0.6931471805599453
SQRT2 = 1.4142135623730951
NEG_INF = float("-inf")


def _vlog(x):
    bits = lax.bitcast_convert_type(x, jnp.int32)
    e = jnp.right_shift(bits, 23) & 0xFF
    ef = (e - 127).astype(jnp.float32)
    m = lax.bitcast_convert_type((bits & 0x007FFFFF) | 0x3F800000, jnp.float32)
    big = m > SQRT2
    m = jnp.where(big, m * 0.5, m)
    ef = jnp.where(big, ef + 1.0, ef)
    r = (m - 1.0) / (m + 1.0)
    r2 = r * r
    p = r2 * (1.0 / 7.0) + (1.0 / 5.0)
    p = p * r2 + (1.0 / 3.0)
    p = p * r2 + 1.0
    return ef * LN2 + 2.0 * r * p


def _iou16(ax1, ay1, ax2, ay2, aarea, bx1, by1, bx2, by2, barea):
    ltx = jnp.maximum(ax1, bx1)
    lty = jnp.maximum(ay1, by1)
    rbx = jnp.minimum(ax2, bx2)
    rby = jnp.minimum(ay2, by2)
    w = jnp.maximum(rbx - ltx, 0.0)
    h = jnp.maximum(rby - lty, 0.0)
    inter = w * h
    union = jnp.maximum(aarea + barea - inter, EPS)
    return inter / union


def _nms_body(x1_h, y1_h, x2_h, y2_h, sc_h, g_h,
              gx1_h, gy1_h, gx2_h, gy2_h, out_h,
              vx1, vy1, vx2, vy2, vsc, vg, varea, vact,
              vgx1, vgy1, vgx2, vgy2, vgarea, vgt, vrec, vout):
    cid = lax.axis_index("c")
    sid = lax.axis_index("s")

    @pl.when((cid == 0) & (sid == 0))
    def _():
        iota = lax.iota(jnp.int32, 16)
        lane0 = iota == 0

        pltpu.sync_copy(x1_h, vx1)
        pltpu.sync_copy(y1_h, vy1)
        pltpu.sync_copy(x2_h, vx2)
        pltpu.sync_copy(y2_h, vy2)
        pltpu.sync_copy(sc_h, vsc)
        pltpu.sync_copy(g_h, vg)
        pltpu.sync_copy(gx1_h, vgx1)
        pltpu.sync_copy(gy1_h, vgy1)
        pltpu.sync_copy(gx2_h, vgx2)
        pltpu.sync_copy(gy2_h, vgy2)

        for c in range(GP // 16):
            sl = pl.ds(c * 16, 16)
            vgarea[sl] = (vgx2[sl] - vgx1[sl]) * (vgy2[sl] - vgy1[sl])
            vrec[sl] = jnp.full((16,), -1, jnp.int32)

        def gt_body(c, carry):
            idx = c * 16 + iota
            r = idx // G
            cc = idx - r * G
            rx1 = plsc.load_gather(vgx1, [r])
            ry1 = plsc.load_gather(vgy1, [r])
            rx2 = plsc.load_gather(vgx2, [r])
            ry2 = plsc.load_gather(vgy2, [r])
            rar = plsc.load_gather(vgarea, [r])
            cx1 = plsc.load_gather(vgx1, [cc])
            cy1 = plsc.load_gather(vgy1, [cc])
            cx2 = plsc.load_gather(vgx2, [cc])
            cy2 = plsc.load_gather(vgy2, [cc])
            car = plsc.load_gather(vgarea, [cc])
            v = _iou16(rx1, ry1, rx2, ry2, rar, cx1, cy1, cx2, cy2, car)
            vgt[pl.ds(c * 16, 16)] = v
            return carry

        lax.fori_loop(0, NGCH, gt_body, 0)

        def init_body(c, carry):
            bm, bidx, cnt = carry
            sl = pl.ds(c * 16, 16)
            area = (vx2[sl] - vx1[sl]) * (vy2[sl] - vy1[sl])
            varea[sl] = area
            act = vg[sl] >= 0
            vact[sl] = jnp.where(act, 1, 0).astype(jnp.int32)
            masked = jnp.where(act, vsc[sl], NEG_INF)
            idxv = c * 16 + iota
            upd = masked >= bm
            bm = jnp.where(upd, masked, bm)
            bidx = jnp.where(upd, idxv, bidx)
            cnt = cnt + jnp.where(act, 1, 0).astype(jnp.int32)
            return bm, bidx, cnt

        bm0 = jnp.full((16,), NEG_INF, jnp.float32)
        bi0 = jnp.full((16,), -1, jnp.int32)
        cv0 = jnp.zeros((16,), jnp.int32)
        bm, bidx, cntv = lax.fori_loop(0, NCH, init_body, (bm0, bi0, cv0))
        top0 = jnp.max(bm)
        i0 = jnp.max(jnp.where(bm == top0, bidx, -1))
        cnt0 = jnp.sum(cntv)

        def cond_fun(state):
            return state[2] > 0

        def body_fun(state):
            i, top, act_cnt, tpull, tpush, pcnt, kcnt = state
            terminal = act_cnt == 1
            iv = jnp.full((16,), i, jnp.int32)
            give = plsc.load_gather(vg, [iv])
            bx1 = plsc.load_gather(vx1, [iv])
            by1 = plsc.load_gather(vy1, [iv])
            bx2 = plsc.load_gather(vx2, [iv])
            by2 = plsc.load_gather(vy2, [iv])
            bar = plsc.load_gather(varea, [iv])

            mi_v = plsc.load_gather(vrec, [give])
            newrec = jnp.where(mi_v < 0, iv, mi_v)
            plsc.store_scatter(vrec, [give], newrec, mask=lane0)
            mi = jnp.max(mi_v)
            has_pull = mi >= 0
            mi_c = jnp.maximum(mi_v, 0)
            mx1 = plsc.load_gather(vx1, [mi_c])
            my1 = plsc.load_gather(vy1, [mi_c])
            mx2 = plsc.load_gather(vx2, [mi_c])
            my2 = plsc.load_gather(vy2, [mi_c])
            mar = plsc.load_gather(varea, [mi_c])
            iou_mi = _iou16(bx1, by1, bx2, by2, bar, mx1, my1, mx2, my2, mar)
            ms = jnp.maximum(iou_mi, EPS)
            pull_v = -_vlog(ms + (1.0 - NMS_THR)) * top
            pull = jnp.max(jnp.where(lane0, pull_v, NEG_INF))
            tpull = tpull + jnp.where(has_pull & jnp.logical_not(terminal),
                                      pull, 0.0)
            pcnt = pcnt + jnp.where(has_pull, 1, 0)

            gbase = give * G

            def sweep(c, carry):
                sbm, sbidx, scnt, spush, skv = carry
                sl = pl.ds(c * 16, 16)
                cx1 = vx1[sl]
                cy1 = vy1[sl]
                cx2 = vx2[sl]
                cy2 = vy2[sl]
                car = varea[sl]
                csc = vsc[sl]
                cg = vg[sl]
                cact = vact[sl] != 0
                idxv = c * 16 + iota
                row = _iou16(bx1, by1, bx2, by2, bar, cx1, cy1, cx2, cy2, car)
                sup = row > NMS_THR
                rest = cact & (idxv != i)
                nact = rest & jnp.logical_not(sup)
                vact[sl] = jnp.where(nact, 1, 0).astype(jnp.int32)
                gtv = plsc.load_gather(vgt, [gbase + jnp.maximum(cg, 0)])
                cand = rest & sup & (cg != give) & (row > gtv)
                lg = _vlog(1.0 - row)
                spush = spush + jnp.where(cand, -lg * csc, 0.0)
                skv = skv + jnp.where(cand, 1, 0).astype(jnp.int32)
                masked = jnp.where(nact, csc, NEG_INF)
                upd = masked >= sbm
                sbm = jnp.where(upd, masked, sbm)
                sbidx = jnp.where(upd, idxv, sbidx)
                scnt = scnt + jnp.where(nact, 1, 0).astype(jnp.int32)
                return sbm, sbidx, scnt, spush, skv

            z_i = jnp.zeros((16,), jnp.int32)
            z_f = jnp.zeros((16,), jnp.float32)
            sbm, sbidx, scnt, spush, skv = lax.fori_loop(
                0, NCH, sweep,
                (jnp.full((16,), NEG_INF, jnp.float32),
                 jnp.full((16,), -1, jnp.int32), z_i, z_f, z_i))

            k = jnp.sum(skv)
            s = jnp.sum(spush)
            tpush = tpush + jnp.where(
                (k > 0) & jnp.logical_not(terminal),
                s / jnp.maximum(k, 1).astype(jnp.float32), 0.0)
            kcnt = kcnt + k
            ntop = jnp.max(sbm)
            ni = jnp.max(jnp.where(sbm == ntop, sbidx, -1))
            ncnt = jnp.sum(scnt)
            return ni, ntop, ncnt, tpull, tpush, pcnt, kcnt

        state0 = (i0, top0, cnt0, jnp.float32(0.0), jnp.float32(0.0),
                  jnp.int32(0), jnp.int32(0))
        (_, _, _, tpull, tpush, pcnt, kcnt) = lax.while_loop(
            cond_fun, body_fun, state0)

        pull_loss = tpull / (pcnt.astype(jnp.float32) + EPS)
        push_loss = tpush / (kcnt.astype(jnp.float32) + EPS)
        res = jnp.where(lane0, push_loss,
                        jnp.where(iota == 1, pull_loss, 0.0))
        vout[...] = res
        pltpu.sync_copy(vout, out_h)


@jax.jit
def _nms_sc(x1, y1, x2, y2, sc, g, gx1, gy1, gx2, gy2):
    mesh = plsc.VectorSubcoreMesh(core_axis_name="c", subcore_axis_name="s")
    f = pl.kernel(
        _nms_body,
        out_type=jax.ShapeDtypeStruct((16,), jnp.float32),
        mesh=mesh,
        scratch_types=[
            pltpu.VMEM((NP,), jnp.float32),
            pltpu.VMEM((NP,), jnp.float32),
            pltpu.VMEM((NP,), jnp.float32),
            pltpu.VMEM((NP,), jnp.float32),
            pltpu.VMEM((NP,), jnp.float32),
            pltpu.VMEM((NP,), jnp.int32),
            pltpu.VMEM((NP,), jnp.float32),
            pltpu.VMEM((NP,), jnp.int32),
            pltpu.VMEM((GP,), jnp.float32),
            pltpu.VMEM((GP,), jnp.float32),
            pltpu.VMEM((GP,), jnp.float32),
            pltpu.VMEM((GP,), jnp.float32),
            pltpu.VMEM((GP,), jnp.float32),
            pltpu.VMEM((GT2P,), jnp.float32),
            pltpu.VMEM((GP,), jnp.int32),
            pltpu.VMEM((16,), jnp.float32),
        ],
    )
    return f(x1, y1, x2, y2, sc, g, gx1, gy1, gx2, gy2)


def kernel(gt_inds, anchor_gt_inds, gt_bboxes, proposal_list):
    del gt_inds
    p = proposal_list.astype(jnp.float32)
    x1 = jnp.pad(p[:, 0], (0, NP - N))
    y1 = jnp.pad(p[:, 1], (0, NP - N))
    x2 = jnp.pad(p[:, 2], (0, NP - N))
    y2 = jnp.pad(p[:, 3], (0, NP - N))
    sc = jnp.pad(p[:, 4], (0, NP - N))
    g = jnp.pad(anchor_gt_inds.astype(jnp.int32), (0, NP - N),
                constant_values=-1)
    gb = gt_bboxes.astype(jnp.float32)
    gx1 = jnp.pad(gb[:, 0], (0, GP - G))
    gy1 = jnp.pad(gb[:, 1], (0, GP - G))
    gx2 = jnp.pad(gb[:, 2], (0, GP - G))
    gy2 = jnp.pad(gb[:, 3], (0, GP - G))
    out = _nms_sc(x1, y1, x2, y2, sc, g, gx1, gy1, gx2, gy2)
    return jnp.stack([out[0], out[1]])


# speedup vs baseline: 19.6530x; 19.6530x over previous
import jax
import jax.numpy as jnp
from jax import lax
from jax.experimental import pallas as pl
from jax.experimental.pallas import tpu as pltpu
from jax.experimental.pallas import tpu_sc as plsc

N = 1000
NP = 1024
NPA = 1040
NCH = NP // 16
G = 50
GP = 80
GSTRIDE = 64
GT2P = G * GSTRIDE + 16
RECP = 80

NMS_THR = 0.5
EPS = 1e-6
LN2 = 0.6931471805599453
SQRT2 = 1.4142135623730951
NEG_INF = float("-inf")


def _vlog(x):
    bits = lax.bitcast_convert_type(x, jnp.int32)
    e = jnp.right_shift(bits, 23) & 0xFF
    ef = (e - 127).astype(jnp.float32)
    m = lax.bitcast_convert_type((bits & 0x007FFFFF) | 0x3F800000, jnp.float32)
    big = m > SQRT2
    m = jnp.where(big, m * 0.5, m)
    ef = jnp.where(big, ef + 1.0, ef)
    r = (m - 1.0) / (m + 1.0)
    r2 = r * r
    p = r2 * (1.0 / 7.0) + (1.0 / 5.0)
    p = p * r2 + (1.0 / 3.0)
    p = p * r2 + 1.0
    return ef * LN2 + 2.0 * r * p


def _bfly_sum(v, iota):
    for d in (1, 2, 4, 8):
        v = v + v[iota ^ d]
    return v


def _bfly_max(v, iota):
    for d in (1, 2, 4, 8):
        v = jnp.maximum(v, v[iota ^ d])
    return v


def _bfly_max_argmax(bm, bidx, iota):
    bmax = _bfly_max(bm, iota)
    sel = jnp.where(bm == bmax, bidx, -1)
    imax = _bfly_max(sel, iota)
    return bmax, imax


def _iou16(ax1, ay1, ax2, ay2, aarea, bx1, by1, bx2, by2, barea):
    ltx = jnp.maximum(ax1, bx1)
    lty = jnp.maximum(ay1, by1)
    rbx = jnp.minimum(ax2, bx2)
    rby = jnp.minimum(ay2, by2)
    w = jnp.maximum(rbx - ltx, 0.0)
    h = jnp.maximum(rby - lty, 0.0)
    inter = w * h
    union = jnp.maximum(aarea + barea - inter, EPS)
    return inter / union


def _nms_body(x1_h, y1_h, x2_h, y2_h, sc_h, g_h,
              gx1_h, gy1_h, gx2_h, gy2_h, out_h,
              vx1, vy1, vx2, vy2, vsc, vg, varea, vact,
              vgx1, vgy1, vgx2, vgy2, vgarea, vgt, vrec, vout, vtmp,
              sti, stf):
    cid = lax.axis_index("c")
    sid = lax.axis_index("s")

    @pl.when((cid == 0) & (sid == 0))
    def _():
        iota = lax.iota(jnp.int32, 16)
        lane0 = iota == 0

        pltpu.sync_copy(x1_h, vx1)
        pltpu.sync_copy(y1_h, vy1)
        pltpu.sync_copy(x2_h, vx2)
        pltpu.sync_copy(y2_h, vy2)
        pltpu.sync_copy(sc_h, vsc)
        pltpu.sync_copy(g_h, vg)
        pltpu.sync_copy(gx1_h, vgx1)
        pltpu.sync_copy(gy1_h, vgy1)
        pltpu.sync_copy(gx2_h, vgx2)
        pltpu.sync_copy(gy2_h, vgy2)

        for c in range(GP // 16):
            sl = pl.ds(c * 16, 16)
            vgarea[sl] = (vgx2[sl] - vgx1[sl]) * (vgy2[sl] - vgy1[sl])
        for c in range(RECP // 16):
            vrec[pl.ds(c * 16, 16)] = jnp.full((16,), -1, jnp.int32)

        def gt_row(r, carry):
            rx1 = jnp.full((16,), vgx1[pl.ds(r, 16)][0], jnp.float32)
            ry1 = jnp.full((16,), vgy1[pl.ds(r, 16)][0], jnp.float32)
            rx2 = jnp.full((16,), vgx2[pl.ds(r, 16)][0], jnp.float32)
            ry2 = jnp.full((16,), vgy2[pl.ds(r, 16)][0], jnp.float32)
            rar = jnp.full((16,), vgarea[pl.ds(r, 16)][0], jnp.float32)
            for c in range(64 // 16):
                sl = pl.ds(c * 16, 16)
                v = _iou16(rx1, ry1, rx2, ry2, rar,
                           vgx1[sl], vgy1[sl], vgx2[sl], vgy2[sl],
                           vgarea[sl])
                vgt[pl.ds(r * GSTRIDE + c * 16, 16)] = v
            return carry

        lax.fori_loop(0, G, gt_row, 0)

        def init_body(c, carry):
            bm, bidx, cnt = carry
            sl = pl.ds(c * 16, 16)
            area = (vx2[sl] - vx1[sl]) * (vy2[sl] - vy1[sl])
            varea[sl] = area
            act = vg[sl] >= 0
            vact[sl] = jnp.where(act, 1, 0).astype(jnp.int32)
            masked = jnp.where(act, vsc[sl], NEG_INF)
            idxv = c * 16 + iota
            upd = masked >= bm
            bm = jnp.where(upd, masked, bm)
            bidx = jnp.where(upd, idxv, bidx)
            cnt = cnt + jnp.where(act, 1, 0).astype(jnp.int32)
            return bm, bidx, cnt

        bm0 = jnp.full((16,), NEG_INF, jnp.float32)
        bi0 = jnp.full((16,), -1, jnp.int32)
        cv0 = jnp.zeros((16,), jnp.int32)
        bm, bidx, cntv = lax.fori_loop(0, NCH, init_body, (bm0, bi0, cv0))
        bm, bidx = _bfly_max_argmax(bm, bidx, iota)
        sti[0] = bidx[0]
        sti[1] = _bfly_sum(cntv, iota)[0]
        sti[2] = 0
        sti[3] = 0
        stf[0] = bm[0]
        stf[1] = 0.0
        stf[2] = 0.0

        def outer(t, carry):
            act_cnt = sti[1]

            @pl.when(act_cnt > 0)
            def _():
                i = sti[0]
                top = stf[0]
                terminal = act_cnt == 1
                sti[4] = vg[pl.ds(i, 16)][0]
                gi = sti[4]
                bx1s = vx1[pl.ds(i, 16)][0]
                by1s = vy1[pl.ds(i, 16)][0]
                bx2s = vx2[pl.ds(i, 16)][0]
                by2s = vy2[pl.ds(i, 16)][0]
                bars = varea[pl.ds(i, 16)][0]
                bx1 = jnp.full((16,), bx1s, jnp.float32)
                by1 = jnp.full((16,), by1s, jnp.float32)
                bx2 = jnp.full((16,), bx2s, jnp.float32)
                by2 = jnp.full((16,), by2s, jnp.float32)
                bar = jnp.full((16,), bars, jnp.float32)

                lane0_o = iota == 0
                rv = vrec[pl.ds(gi, 16)]
                sti[5] = rv[0]
                mi = sti[5]
                newr = jnp.where(mi < 0, i, mi)
                vrec[pl.ds(gi, 16)] = jnp.where(lane0_o, newr, rv)
                has_pull = mi >= 0

                sti[6] = jnp.maximum(mi, 0)
                mic = sti[6]
                mx1 = jnp.full((16,), vx1[pl.ds(mic, 16)][0], jnp.float32)
                my1 = jnp.full((16,), vy1[pl.ds(mic, 16)][0], jnp.float32)
                mx2 = jnp.full((16,), vx2[pl.ds(mic, 16)][0], jnp.float32)
                my2 = jnp.full((16,), vy2[pl.ds(mic, 16)][0], jnp.float32)
                mar = jnp.full((16,), varea[pl.ds(mic, 16)][0], jnp.float32)
                iou_mi = _iou16(bx1, by1, bx2, by2, bar,
                                mx1, my1, mx2, my2, mar)
                ms = jnp.maximum(iou_mi, EPS)
                vtmp[...] = -_vlog(ms + (1.0 - NMS_THR)) * top
                pull = vtmp[...][0]
                add_pull = jnp.where(has_pull, 1, 0) * jnp.where(terminal, 0, 1)
                stf[1] = stf[1] + jnp.where(add_pull != 0, pull, 0.0)
                sti[2] = sti[2] + jnp.where(has_pull, 1, 0)

                gb = gi * GSTRIDE
                gr0 = vgt[pl.ds(gb, 16)]
                gr1 = vgt[pl.ds(gb + 16, 16)]
                gr2 = vgt[pl.ds(gb + 32, 16)]
                gr3 = vgt[pl.ds(gb + 48, 16)]

                def sweep(c, carry):
                    sbm, sbidx, scnt, spush, skv = carry
                    sl = pl.ds(c * 16, 16)
                    csc = vsc[sl]
                    cg = vg[sl]
                    acti = vact[sl]
                    idxv = c * 16 + iota
                    row = _iou16(bx1, by1, bx2, by2, bar,
                                 vx1[sl], vy1[sl], vx2[sl], vy2[sl],
                                 varea[sl])
                    supi = jnp.where(row > NMS_THR, 1, 0).astype(jnp.int32)
                    resti = acti & jnp.where(idxv != i, 1, 0).astype(jnp.int32)
                    nacti = resti & (1 - supi)
                    vact[sl] = nacti
                    cgc = jnp.maximum(cg, 0)
                    lanei = cgc & 15
                    tsel = jnp.right_shift(cgc, 4)
                    gtv = jnp.where(
                        tsel == 0, gr0[lanei],
                        jnp.where(tsel == 1, gr1[lanei],
                                  jnp.where(tsel == 2, gr2[lanei],
                                            gr3[lanei])))
                    candi = (resti & supi
                             & jnp.where(cg != gi, 1, 0).astype(jnp.int32)
                             & jnp.where(row > gtv, 1, 0).astype(jnp.int32))
                    cand = candi != 0
                    lg = _vlog(1.0 - row)
                    spush = spush + jnp.where(cand, -lg * csc, 0.0)
                    skv = skv + candi
                    nact = nacti != 0
                    masked = jnp.where(nact, csc, NEG_INF)
                    upd = masked >= sbm
                    sbm = jnp.where(upd, masked, sbm)
                    sbidx = jnp.where(upd, idxv, sbidx)
                    scnt = scnt + nacti
                    return sbm, sbidx, scnt, spush, skv

                z_i = jnp.zeros((16,), jnp.int32)
                z_f = jnp.zeros((16,), jnp.float32)
                sbm, sbidx, scnt, spush, skv = lax.fori_loop(
                    0, NCH, sweep,
                    (jnp.full((16,), NEG_INF, jnp.float32),
                     jnp.full((16,), -1, jnp.int32), z_i, z_f, z_i))

                k = _bfly_sum(skv, iota)[0]
                s = _bfly_sum(spush, iota)[0]
                add_push = jnp.where(k > 0, 1, 0) * jnp.where(terminal, 0, 1)
                kf = jnp.maximum(k, 1).astype(jnp.float32)
                vtmp[...] = jnp.full((16,), s, jnp.float32) / jnp.full(
                    (16,), kf, jnp.float32)
                stf[2] = stf[2] + jnp.where(add_push != 0, vtmp[...][0], 0.0)
                sti[3] = sti[3] + k
                sbm, sbidx = _bfly_max_argmax(sbm, sbidx, iota)
                sti[0] = sbidx[0]
                stf[0] = sbm[0]
                sti[1] = _bfly_sum(scnt, iota)[0]

            return carry

        lax.fori_loop(0, N, outer, 0)

        pull_loss = jnp.full((16,), stf[1], jnp.float32) / jnp.full(
            (16,), sti[2].astype(jnp.float32) + EPS, jnp.float32)
        push_loss = jnp.full((16,), stf[2], jnp.float32) / jnp.full(
            (16,), sti[3].astype(jnp.float32) + EPS, jnp.float32)
        res = jnp.where(lane0, push_loss,
                        jnp.where(iota == 1, pull_loss, 0.0))
        vout[...] = res
        pltpu.sync_copy(vout, out_h)


@jax.jit
def _nms_sc(x1, y1, x2, y2, sc, g, gx1, gy1, gx2, gy2):
    mesh = plsc.VectorSubcoreMesh(core_axis_name="c", subcore_axis_name="s")
    f = pl.kernel(
        _nms_body,
        out_type=jax.ShapeDtypeStruct((16,), jnp.float32),
        mesh=mesh,
        scratch_types=[
            pltpu.VMEM((NPA,), jnp.float32),
            pltpu.VMEM((NPA,), jnp.float32),
            pltpu.VMEM((NPA,), jnp.float32),
            pltpu.VMEM((NPA,), jnp.float32),
            pltpu.VMEM((NPA,), jnp.float32),
            pltpu.VMEM((NPA,), jnp.int32),
            pltpu.VMEM((NPA,), jnp.float32),
            pltpu.VMEM((NP,), jnp.int32),
            pltpu.VMEM((GP,), jnp.float32),
            pltpu.VMEM((GP,), jnp.float32),
            pltpu.VMEM((GP,), jnp.float32),
            pltpu.VMEM((GP,), jnp.float32),
            pltpu.VMEM((GP,), jnp.float32),
            pltpu.VMEM((GT2P,), jnp.float32),
            pltpu.VMEM((RECP,), jnp.int32),
            pltpu.VMEM((16,), jnp.float32),
            pltpu.VMEM((16,), jnp.float32),
            pltpu.SMEM((8,), jnp.int32),
            pltpu.SMEM((8,), jnp.float32),
        ],
    )
    return f(x1, y1, x2, y2, sc, g, gx1, gy1, gx2, gy2)


def kernel(gt_inds, anchor_gt_inds, gt_bboxes, proposal_list):
    del gt_inds
    p = proposal_list.astype(jnp.float32)
    x1 = jnp.pad(p[:, 0], (0, NPA - N))
    y1 = jnp.pad(p[:, 1], (0, NPA - N))
    x2 = jnp.pad(p[:, 2], (0, NPA - N))
    y2 = jnp.pad(p[:, 3], (0, NPA - N))
    sc = jnp.pad(p[:, 4], (0, NPA - N))
    g = jnp.pad(anchor_gt_inds.astype(jnp.int32), (0, NPA - N),
                constant_values=-1)
    gb = gt_bboxes.astype(jnp.float32)
    gx1 = jnp.pad(gb[:, 0], (0, GP - G))
    gy1 = jnp.pad(gb[:, 1], (0, GP - G))
    gx2 = jnp.pad(gb[:, 2], (0, GP - G))
    gy2 = jnp.pad(gb[:, 3], (0, GP - G))
    out = _nms_sc(x1, y1, x2, y2, sc, g, gx1, gy1, gx2, gy2)
    return jnp.stack([out[0], out[1]])


# speedup vs baseline: 22.4133x; 1.1405x over previous
import jax
import jax.numpy as jnp
from jax import lax
from jax.experimental import pallas as pl
from jax.experimental.pallas import tpu as pltpu
from jax.experimental.pallas import tpu_sc as plsc

N = 1000
NP = 1024
NPA = 1040
NCH = NP // 16
G = 50
GP = 80
GSTRIDE = 64
GT2P = G * GSTRIDE + 16
RECP = 80

NMS_THR = 0.5
EPS = 1e-6
LN2 = 0.6931471805599453
SQRT2 = 1.4142135623730951
NEG_INF = float("-inf")


def _vlog(x):
    bits = lax.bitcast_convert_type(x, jnp.int32)
    e = jnp.right_shift(bits, 23) & 0xFF
    ef = (e - 127).astype(jnp.float32)
    m = lax.bitcast_convert_type((bits & 0x007FFFFF) | 0x3F800000, jnp.float32)
    r = (m - 1.0) / (m + 1.0)
    r2 = r * r
    p = r2 * (1.0 / 7.0) + (1.0 / 5.0)
    p = p * r2 + (1.0 / 3.0)
    p = p * r2 + 1.0
    return ef * LN2 + 2.0 * r * p


def _bfly_sum(v, iota):
    for d in (1, 2, 4, 8):
        v = v + v[iota ^ d]
    return v


def _bfly_max(v, iota):
    for d in (1, 2, 4, 8):
        v = jnp.maximum(v, v[iota ^ d])
    return v


def _bfly_max_argmax(bm, bidx, iota):
    bmax = _bfly_max(bm, iota)
    sel = jnp.where(bm == bmax, bidx, -1)
    imax = _bfly_max(sel, iota)
    return bmax, imax


def _iou16(ax1, ay1, ax2, ay2, aarea, bx1, by1, bx2, by2, barea):
    ltx = jnp.maximum(ax1, bx1)
    lty = jnp.maximum(ay1, by1)
    rbx = jnp.minimum(ax2, bx2)
    rby = jnp.minimum(ay2, by2)
    w = jnp.maximum(rbx - ltx, 0.0)
    h = jnp.maximum(rby - lty, 0.0)
    inter = w * h
    union = jnp.maximum(aarea + barea - inter, EPS)
    return inter / union


def _nms_body(x1_h, y1_h, x2_h, y2_h, sc_h, g_h,
              gx1_h, gy1_h, gx2_h, gy2_h, out_h,
              vx1, vy1, vx2, vy2, vsc, vg, varea, vact,
              vgx1, vgy1, vgx2, vgy2, vgarea, vgt, vrec, vout, vtmp,
              sti, stf):
    cid = lax.axis_index("c")
    sid = lax.axis_index("s")

    @pl.when((cid == 0) & (sid == 0))
    def _():
        iota = lax.iota(jnp.int32, 16)
        lane0 = iota == 0

        pltpu.sync_copy(x1_h, vx1)
        pltpu.sync_copy(y1_h, vy1)
        pltpu.sync_copy(x2_h, vx2)
        pltpu.sync_copy(y2_h, vy2)
        pltpu.sync_copy(sc_h, vsc)
        pltpu.sync_copy(g_h, vg)
        pltpu.sync_copy(gx1_h, vgx1)
        pltpu.sync_copy(gy1_h, vgy1)
        pltpu.sync_copy(gx2_h, vgx2)
        pltpu.sync_copy(gy2_h, vgy2)

        for c in range(GP // 16):
            sl = pl.ds(c * 16, 16)
            vgarea[sl] = (vgx2[sl] - vgx1[sl]) * (vgy2[sl] - vgy1[sl])
        for c in range(RECP // 16):
            vrec[pl.ds(c * 16, 16)] = jnp.full((16,), -1, jnp.int32)

        def gt_row(r, carry):
            rx1 = jnp.full((16,), vgx1[pl.ds(r, 16)][0], jnp.float32)
            ry1 = jnp.full((16,), vgy1[pl.ds(r, 16)][0], jnp.float32)
            rx2 = jnp.full((16,), vgx2[pl.ds(r, 16)][0], jnp.float32)
            ry2 = jnp.full((16,), vgy2[pl.ds(r, 16)][0], jnp.float32)
            rar = jnp.full((16,), vgarea[pl.ds(r, 16)][0], jnp.float32)
            for c in range(64 // 16):
                sl = pl.ds(c * 16, 16)
                v = _iou16(rx1, ry1, rx2, ry2, rar,
                           vgx1[sl], vgy1[sl], vgx2[sl], vgy2[sl],
                           vgarea[sl])
                vgt[pl.ds(r * GSTRIDE + c * 16, 16)] = v
            return carry

        lax.fori_loop(0, G, gt_row, 0)

        def init_body(c, carry):
            bm, bidx, cnt = carry
            sl = pl.ds(c * 16, 16)
            area = (vx2[sl] - vx1[sl]) * (vy2[sl] - vy1[sl])
            varea[sl] = area
            act = vg[sl] >= 0
            vact[sl] = jnp.where(act, 1, 0).astype(jnp.int32)
            masked = jnp.where(act, vsc[sl], NEG_INF)
            idxv = c * 16 + iota
            upd = masked >= bm
            bm = jnp.where(upd, masked, bm)
            bidx = jnp.where(upd, idxv, bidx)
            cnt = cnt + jnp.where(act, 1, 0).astype(jnp.int32)
            return bm, bidx, cnt

        bm0 = jnp.full((16,), NEG_INF, jnp.float32)
        bi0 = jnp.full((16,), -1, jnp.int32)
        cv0 = jnp.zeros((16,), jnp.int32)
        bm, bidx, cntv = lax.fori_loop(0, NCH, init_body, (bm0, bi0, cv0))
        bm, bidx = _bfly_max_argmax(bm, bidx, iota)
        sti[0] = bidx[0]
        sti[1] = _bfly_sum(cntv, iota)[0]
        sti[2] = 0
        sti[3] = 0
        stf[0] = bm[0]
        stf[1] = 0.0
        stf[2] = 0.0

        def outer(t, carry):
            act_cnt = sti[1]

            @pl.when(act_cnt > 0)
            def _():
                i = sti[0]
                top = stf[0]
                terminal = act_cnt == 1
                lane0_c = iota == 0
                av = vact[pl.ds(i, 16)]
                vact[pl.ds(i, 16)] = jnp.where(lane0_c, 0, av)
                sti[4] = vg[pl.ds(i, 16)][0]
                gi = sti[4]
                bx1s = vx1[pl.ds(i, 16)][0]
                by1s = vy1[pl.ds(i, 16)][0]
                bx2s = vx2[pl.ds(i, 16)][0]
                by2s = vy2[pl.ds(i, 16)][0]
                bars = varea[pl.ds(i, 16)][0]
                bx1 = jnp.full((16,), bx1s, jnp.float32)
                by1 = jnp.full((16,), by1s, jnp.float32)
                bx2 = jnp.full((16,), bx2s, jnp.float32)
                by2 = jnp.full((16,), by2s, jnp.float32)
                bar = jnp.full((16,), bars, jnp.float32)

                lane0_o = iota == 0
                rv = vrec[pl.ds(gi, 16)]
                sti[5] = rv[0]
                mi = sti[5]
                newr = jnp.where(mi < 0, i, mi)
                vrec[pl.ds(gi, 16)] = jnp.where(lane0_o, newr, rv)
                has_pull = mi >= 0

                sti[6] = jnp.maximum(mi, 0)
                mic = sti[6]
                mx1 = jnp.full((16,), vx1[pl.ds(mic, 16)][0], jnp.float32)
                my1 = jnp.full((16,), vy1[pl.ds(mic, 16)][0], jnp.float32)
                mx2 = jnp.full((16,), vx2[pl.ds(mic, 16)][0], jnp.float32)
                my2 = jnp.full((16,), vy2[pl.ds(mic, 16)][0], jnp.float32)
                mar = jnp.full((16,), varea[pl.ds(mic, 16)][0], jnp.float32)
                iou_mi = _iou16(bx1, by1, bx2, by2, bar,
                                mx1, my1, mx2, my2, mar)
                ms = jnp.maximum(iou_mi, EPS)
                vtmp[...] = -_vlog(ms + (1.0 - NMS_THR)) * top
                pull = vtmp[...][0]
                add_pull = jnp.where(has_pull, 1, 0) * jnp.where(terminal, 0, 1)
                stf[1] = stf[1] + jnp.where(add_pull != 0, pull, 0.0)
                sti[2] = sti[2] + jnp.where(has_pull, 1, 0)

                gb = gi * GSTRIDE
                gr0 = vgt[pl.ds(gb, 16)]
                gr1 = vgt[pl.ds(gb + 16, 16)]
                gr2 = vgt[pl.ds(gb + 32, 16)]
                gr3 = vgt[pl.ds(gb + 48, 16)]

                def sweep(c, carry):
                    sbm, sbidx, scnt, spush, skv = carry
                    sl = pl.ds(c * 16, 16)
                    csc = vsc[sl]
                    cg = vg[sl]
                    acti = vact[sl]
                    idxv = c * 16 + iota
                    row = _iou16(bx1, by1, bx2, by2, bar,
                                 vx1[sl], vy1[sl], vx2[sl], vy2[sl],
                                 varea[sl])
                    supi = jnp.where(row > NMS_THR, 1, 0).astype(jnp.int32)
                    resti = acti
                    nacti = resti & (1 - supi)
                    vact[sl] = nacti
                    cgc = jnp.maximum(cg, 0)
                    lanei = cgc & 15
                    tsel = jnp.right_shift(cgc, 4)
                    gtv = jnp.where(
                        tsel == 0, gr0[lanei],
                        jnp.where(tsel == 1, gr1[lanei],
                                  jnp.where(tsel == 2, gr2[lanei],
                                            gr3[lanei])))
                    candi = (resti & supi
                             & jnp.where(cg != gi, 1, 0).astype(jnp.int32)
                             & jnp.where(row > gtv, 1, 0).astype(jnp.int32))
                    cand = candi != 0
                    lg = _vlog(1.0 - row)
                    spush = spush + jnp.where(cand, -lg * csc, 0.0)
                    skv = skv + candi
                    nact = nacti != 0
                    masked = jnp.where(nact, csc, NEG_INF)
                    upd = masked >= sbm
                    sbm = jnp.where(upd, masked, sbm)
                    sbidx = jnp.where(upd, idxv, sbidx)
                    scnt = scnt + nacti
                    return sbm, sbidx, scnt, spush, skv

                z_i = jnp.zeros((16,), jnp.int32)
                z_f = jnp.zeros((16,), jnp.float32)
                sbm, sbidx, scnt, spush, skv = lax.fori_loop(
                    0, NCH, sweep,
                    (jnp.full((16,), NEG_INF, jnp.float32),
                     jnp.full((16,), -1, jnp.int32), z_i, z_f, z_i))

                k = _bfly_sum(skv, iota)[0]
                s = _bfly_sum(spush, iota)[0]
                add_push = jnp.where(k > 0, 1, 0) * jnp.where(terminal, 0, 1)
                kf = jnp.maximum(k, 1).astype(jnp.float32)
                vtmp[...] = jnp.full((16,), s, jnp.float32) / jnp.full(
                    (16,), kf, jnp.float32)
                stf[2] = stf[2] + jnp.where(add_push != 0, vtmp[...][0], 0.0)
                sti[3] = sti[3] + k
                sbm, sbidx = _bfly_max_argmax(sbm, sbidx, iota)
                sti[0] = sbidx[0]
                stf[0] = sbm[0]
                sti[1] = _bfly_sum(scnt, iota)[0]

            return carry

        def outer_blk(b, carry):
            @pl.when(sti[1] > 0)
            def _():
                lax.fori_loop(0, 16, outer, 0)
            return carry

        lax.fori_loop(0, NP // 16, outer_blk, 0)

        pull_loss = jnp.full((16,), stf[1], jnp.float32) / jnp.full(
            (16,), sti[2].astype(jnp.float32) + EPS, jnp.float32)
        push_loss = jnp.full((16,), stf[2], jnp.float32) / jnp.full(
            (16,), sti[3].astype(jnp.float32) + EPS, jnp.float32)
        res = jnp.where(lane0, push_loss,
                        jnp.where(iota == 1, pull_loss, 0.0))
        vout[...] = res
        pltpu.sync_copy(vout, out_h)


@jax.jit
def _nms_sc(x1, y1, x2, y2, sc, g, gx1, gy1, gx2, gy2):
    mesh = plsc.VectorSubcoreMesh(core_axis_name="c", subcore_axis_name="s")
    f = pl.kernel(
        _nms_body,
        out_type=jax.ShapeDtypeStruct((16,), jnp.float32),
        mesh=mesh,
        scratch_types=[
            pltpu.VMEM((NPA,), jnp.float32),
            pltpu.VMEM((NPA,), jnp.float32),
            pltpu.VMEM((NPA,), jnp.float32),
            pltpu.VMEM((NPA,), jnp.float32),
            pltpu.VMEM((NPA,), jnp.float32),
            pltpu.VMEM((NPA,), jnp.int32),
            pltpu.VMEM((NPA,), jnp.float32),
            pltpu.VMEM((NPA,), jnp.int32),
            pltpu.VMEM((GP,), jnp.float32),
            pltpu.VMEM((GP,), jnp.float32),
            pltpu.VMEM((GP,), jnp.float32),
            pltpu.VMEM((GP,), jnp.float32),
            pltpu.VMEM((GP,), jnp.float32),
            pltpu.VMEM((GT2P,), jnp.float32),
            pltpu.VMEM((RECP,), jnp.int32),
            pltpu.VMEM((16,), jnp.float32),
            pltpu.VMEM((16,), jnp.float32),
            pltpu.SMEM((8,), jnp.int32),
            pltpu.SMEM((8,), jnp.float32),
        ],
    )
    return f(x1, y1, x2, y2, sc, g, gx1, gy1, gx2, gy2)


def kernel(gt_inds, anchor_gt_inds, gt_bboxes, proposal_list):
    del gt_inds
    p = proposal_list.astype(jnp.float32)
    x1 = jnp.pad(p[:, 0], (0, NPA - N))
    y1 = jnp.pad(p[:, 1], (0, NPA - N))
    x2 = jnp.pad(p[:, 2], (0, NPA - N))
    y2 = jnp.pad(p[:, 3], (0, NPA - N))
    sc = jnp.pad(p[:, 4], (0, NPA - N))
    g = jnp.pad(anchor_gt_inds.astype(jnp.int32), (0, NPA - N),
                constant_values=-1)
    gb = gt_bboxes.astype(jnp.float32)
    gx1 = jnp.pad(gb[:, 0], (0, GP - G))
    gy1 = jnp.pad(gb[:, 1], (0, GP - G))
    gx2 = jnp.pad(gb[:, 2], (0, GP - G))
    gy2 = jnp.pad(gb[:, 3], (0, GP - G))
    out = _nms_sc(x1, y1, x2, y2, sc, g, gx1, gy1, gx2, gy2)
    return jnp.stack([out[0], out[1]])


# speedup vs baseline: 23.6605x; 1.0556x over previous
import jax
import jax.numpy as jnp
from jax import lax
from jax.experimental import pallas as pl
from jax.experimental.pallas import tpu as pltpu
from jax.experimental.pallas import tpu_sc as plsc

N = 1000
NP = 1024
NPA = 1040
NCH = NP // 16
G = 50
GP = 80
GSTRIDE = 64
GT2P = G * GSTRIDE + 16
RECP = 80

NMS_THR = 0.5
EPS = 1e-6
LN2 = 0.6931471805599453
SQRT2 = 1.4142135623730951
NEG_INF = float("-inf")


def _vlog(x):
    bits = lax.bitcast_convert_type(x, jnp.int32)
    e = jnp.right_shift(bits, 23) & 0xFF
    ef = (e - 127).astype(jnp.float32)
    m = lax.bitcast_convert_type((bits & 0x007FFFFF) | 0x3F800000, jnp.float32)
    r = (m - 1.0) / (m + 1.0)
    r2 = r * r
    p = r2 * (1.0 / 7.0) + (1.0 / 5.0)
    p = p * r2 + (1.0 / 3.0)
    p = p * r2 + 1.0
    return ef * LN2 + 2.0 * r * p


def _bfly_sum(v, iota):
    for d in (1, 2, 4, 8):
        v = v + v[iota ^ d]
    return v


def _bfly_max(v, iota):
    for d in (1, 2, 4, 8):
        v = jnp.maximum(v, v[iota ^ d])
    return v


def _bfly_max_argmax(bm, bidx, iota):
    bmax = _bfly_max(bm, iota)
    sel = jnp.where(bm == bmax, bidx, -1)
    imax = _bfly_max(sel, iota)
    return bmax, imax


def _iou16(ax1, ay1, ax2, ay2, aarea, bx1, by1, bx2, by2, barea):
    ltx = jnp.maximum(ax1, bx1)
    lty = jnp.maximum(ay1, by1)
    rbx = jnp.minimum(ax2, bx2)
    rby = jnp.minimum(ay2, by2)
    w = jnp.maximum(rbx - ltx, 0.0)
    h = jnp.maximum(rby - lty, 0.0)
    inter = w * h
    union = jnp.maximum(aarea + barea - inter, EPS)
    return inter / union


def _nms_body(x1_h, y1_h, x2_h, y2_h, sc_h, g_h,
              gx1_h, gy1_h, gx2_h, gy2_h, out_h,
              vx1, vy1, vx2, vy2, vsc, vg, varea, vact,
              vgx1, vgy1, vgx2, vgy2, vgarea, vgt, vrec, vout, vtmp,
              sti, stf, sem):
    cid = lax.axis_index("c")
    sid = lax.axis_index("s")

    @pl.when((cid == 0) & (sid == 0))
    def _():
        iota = lax.iota(jnp.int32, 16)
        lane0 = iota == 0

        copies = [pltpu.async_copy(s, d, sem) for s, d in (
            (x1_h, vx1), (y1_h, vy1), (x2_h, vx2), (y2_h, vy2),
            (sc_h, vsc), (g_h, vg), (gx1_h, vgx1), (gy1_h, vgy1),
            (gx2_h, vgx2), (gy2_h, vgy2))]
        for c in copies:
            c.wait()

        for c in range(GP // 16):
            sl = pl.ds(c * 16, 16)
            vgarea[sl] = (vgx2[sl] - vgx1[sl]) * (vgy2[sl] - vgy1[sl])
        for c in range(RECP // 16):
            vrec[pl.ds(c * 16, 16)] = jnp.full((16,), -1, jnp.int32)

        def gt_row(r, carry):
            rx1 = jnp.full((16,), vgx1[pl.ds(r, 16)][0], jnp.float32)
            ry1 = jnp.full((16,), vgy1[pl.ds(r, 16)][0], jnp.float32)
            rx2 = jnp.full((16,), vgx2[pl.ds(r, 16)][0], jnp.float32)
            ry2 = jnp.full((16,), vgy2[pl.ds(r, 16)][0], jnp.float32)
            rar = jnp.full((16,), vgarea[pl.ds(r, 16)][0], jnp.float32)
            for c in range(64 // 16):
                sl = pl.ds(c * 16, 16)
                v = _iou16(rx1, ry1, rx2, ry2, rar,
                           vgx1[sl], vgy1[sl], vgx2[sl], vgy2[sl],
                           vgarea[sl])
                vgt[pl.ds(r * GSTRIDE + c * 16, 16)] = v
            return carry

        lax.fori_loop(0, G, gt_row, 0)

        def init_body(c, carry):
            bm, bidx, cnt = carry
            sl = pl.ds(c * 16, 16)
            area = (vx2[sl] - vx1[sl]) * (vy2[sl] - vy1[sl])
            varea[sl] = area
            act = vg[sl] >= 0
            vact[sl] = jnp.where(act, 1, 0).astype(jnp.int32)
            masked = jnp.where(act, vsc[sl], NEG_INF)
            idxv = c * 16 + iota
            upd = masked >= bm
            bm = jnp.where(upd, masked, bm)
            bidx = jnp.where(upd, idxv, bidx)
            cnt = cnt + jnp.where(act, 1, 0).astype(jnp.int32)
            return bm, bidx, cnt

        bm0 = jnp.full((16,), NEG_INF, jnp.float32)
        bi0 = jnp.full((16,), -1, jnp.int32)
        cv0 = jnp.zeros((16,), jnp.int32)
        bm, bidx, cntv = lax.fori_loop(0, NCH, init_body, (bm0, bi0, cv0))
        bm, bidx = _bfly_max_argmax(bm, bidx, iota)
        sti[0] = bidx[0]
        sti[1] = _bfly_sum(cntv, iota)[0]
        sti[2] = 0
        sti[3] = 0
        stf[0] = bm[0]
        stf[1] = 0.0
        stf[2] = 0.0

        def outer(t, carry):
            act_cnt = sti[1]

            @pl.when(act_cnt > 0)
            def _():
                i = sti[0]
                top = stf[0]
                terminal = act_cnt == 1
                lane0_c = iota == 0
                av = vact[pl.ds(i, 16)]
                vact[pl.ds(i, 16)] = jnp.where(lane0_c, 0, av)
                sti[4] = vg[pl.ds(i, 16)][0]
                gi = sti[4]
                bx1s = vx1[pl.ds(i, 16)][0]
                by1s = vy1[pl.ds(i, 16)][0]
                bx2s = vx2[pl.ds(i, 16)][0]
                by2s = vy2[pl.ds(i, 16)][0]
                bars = varea[pl.ds(i, 16)][0]
                bx1 = jnp.full((16,), bx1s, jnp.float32)
                by1 = jnp.full((16,), by1s, jnp.float32)
                bx2 = jnp.full((16,), bx2s, jnp.float32)
                by2 = jnp.full((16,), by2s, jnp.float32)
                bar = jnp.full((16,), bars, jnp.float32)

                lane0_o = iota == 0
                rv = vrec[pl.ds(gi, 16)]
                sti[5] = rv[0]
                mi = sti[5]
                newr = jnp.where(mi < 0, i, mi)
                vrec[pl.ds(gi, 16)] = jnp.where(lane0_o, newr, rv)
                has_pull = mi >= 0

                sti[6] = jnp.maximum(mi, 0)
                mic = sti[6]
                mx1 = jnp.full((16,), vx1[pl.ds(mic, 16)][0], jnp.float32)
                my1 = jnp.full((16,), vy1[pl.ds(mic, 16)][0], jnp.float32)
                mx2 = jnp.full((16,), vx2[pl.ds(mic, 16)][0], jnp.float32)
                my2 = jnp.full((16,), vy2[pl.ds(mic, 16)][0], jnp.float32)
                mar = jnp.full((16,), varea[pl.ds(mic, 16)][0], jnp.float32)
                iou_mi = _iou16(bx1, by1, bx2, by2, bar,
                                mx1, my1, mx2, my2, mar)
                ms = jnp.maximum(iou_mi, EPS)
                vtmp[...] = -_vlog(ms + (1.0 - NMS_THR)) * top
                pull = vtmp[...][0]
                add_pull = jnp.where(has_pull, 1, 0) * jnp.where(terminal, 0, 1)
                stf[1] = stf[1] + jnp.where(add_pull != 0, pull, 0.0)
                sti[2] = sti[2] + jnp.where(has_pull, 1, 0)

                gb = gi * GSTRIDE
                gr0 = vgt[pl.ds(gb, 16)]
                gr1 = vgt[pl.ds(gb + 16, 16)]
                gr2 = vgt[pl.ds(gb + 32, 16)]
                gr3 = vgt[pl.ds(gb + 48, 16)]

                def sweep(c, carry):
                    sbm, sbidx, scnt, spush, skv = carry
                    sl = pl.ds(c * 16, 16)
                    csc = vsc[sl]
                    cg = vg[sl]
                    acti = vact[sl]
                    idxv = c * 16 + iota
                    row = _iou16(bx1, by1, bx2, by2, bar,
                                 vx1[sl], vy1[sl], vx2[sl], vy2[sl],
                                 varea[sl])
                    supi = jnp.where(row > NMS_THR, 1, 0).astype(jnp.int32)
                    resti = acti
                    nacti = resti & (1 - supi)
                    vact[sl] = nacti
                    cgc = jnp.maximum(cg, 0)
                    lanei = cgc & 15
                    tsel = jnp.right_shift(cgc, 4)
                    gtv = jnp.where(
                        tsel == 0, gr0[lanei],
                        jnp.where(tsel == 1, gr1[lanei],
                                  jnp.where(tsel == 2, gr2[lanei],
                                            gr3[lanei])))
                    candi = (resti & supi
                             & jnp.where(cg != gi, 1, 0).astype(jnp.int32)
                             & jnp.where(row > gtv, 1, 0).astype(jnp.int32))
                    cand = candi != 0
                    lg = _vlog(1.0 - row)
                    spush = spush + jnp.where(cand, -lg * csc, 0.0)
                    skv = skv + candi
                    nact = nacti != 0
                    masked = jnp.where(nact, csc, NEG_INF)
                    upd = masked >= sbm
                    sbm = jnp.where(upd, masked, sbm)
                    sbidx = jnp.where(upd, idxv, sbidx)
                    scnt = scnt + nacti
                    return sbm, sbidx, scnt, spush, skv

                z_i = jnp.zeros((16,), jnp.int32)
                z_f = jnp.zeros((16,), jnp.float32)
                sbm, sbidx, scnt, spush, skv = lax.fori_loop(
                    0, NCH, sweep,
                    (jnp.full((16,), NEG_INF, jnp.float32),
                     jnp.full((16,), -1, jnp.int32), z_i, z_f, z_i))

                k = _bfly_sum(skv, iota)[0]
                s = _bfly_sum(spush, iota)[0]
                add_push = jnp.where(k > 0, 1, 0) * jnp.where(terminal, 0, 1)
                kf = jnp.maximum(k, 1).astype(jnp.float32)
                vtmp[...] = jnp.full((16,), s, jnp.float32) / jnp.full(
                    (16,), kf, jnp.float32)
                stf[2] = stf[2] + jnp.where(add_push != 0, vtmp[...][0], 0.0)
                sti[3] = sti[3] + k
                sbm, sbidx = _bfly_max_argmax(sbm, sbidx, iota)
                sti[0] = sbidx[0]
                stf[0] = sbm[0]
                sti[1] = _bfly_sum(scnt, iota)[0]

            return carry

        def outer_blk(b, carry):
            @pl.when(sti[1] > 0)
            def _():
                lax.fori_loop(0, 16, outer, 0)
            return carry

        lax.fori_loop(0, NP // 16, outer_blk, 0)

        pull_loss = jnp.full((16,), stf[1], jnp.float32) / jnp.full(
            (16,), sti[2].astype(jnp.float32) + EPS, jnp.float32)
        push_loss = jnp.full((16,), stf[2], jnp.float32) / jnp.full(
            (16,), sti[3].astype(jnp.float32) + EPS, jnp.float32)
        res = jnp.where(lane0, push_loss,
                        jnp.where(iota == 1, pull_loss, 0.0))
        vout[...] = res
        pltpu.sync_copy(vout, out_h)


@jax.jit
def _nms_sc(x1, y1, x2, y2, sc, g, gx1, gy1, gx2, gy2):
    mesh = plsc.VectorSubcoreMesh(core_axis_name="c", subcore_axis_name="s")
    f = pl.kernel(
        _nms_body,
        out_type=jax.ShapeDtypeStruct((16,), jnp.float32),
        mesh=mesh,
        scratch_types=[
            pltpu.VMEM((NPA,), jnp.float32),
            pltpu.VMEM((NPA,), jnp.float32),
            pltpu.VMEM((NPA,), jnp.float32),
            pltpu.VMEM((NPA,), jnp.float32),
            pltpu.VMEM((NPA,), jnp.float32),
            pltpu.VMEM((NPA,), jnp.int32),
            pltpu.VMEM((NPA,), jnp.float32),
            pltpu.VMEM((NPA,), jnp.int32),
            pltpu.VMEM((GP,), jnp.float32),
            pltpu.VMEM((GP,), jnp.float32),
            pltpu.VMEM((GP,), jnp.float32),
            pltpu.VMEM((GP,), jnp.float32),
            pltpu.VMEM((GP,), jnp.float32),
            pltpu.VMEM((GT2P,), jnp.float32),
            pltpu.VMEM((RECP,), jnp.int32),
            pltpu.VMEM((16,), jnp.float32),
            pltpu.VMEM((16,), jnp.float32),
            pltpu.SMEM((8,), jnp.int32),
            pltpu.SMEM((8,), jnp.float32),
            pltpu.SemaphoreType.DMA,
        ],
    )
    return f(x1, y1, x2, y2, sc, g, gx1, gy1, gx2, gy2)


def kernel(gt_inds, anchor_gt_inds, gt_bboxes, proposal_list):
    del gt_inds
    p = proposal_list.astype(jnp.float32)
    x1 = jnp.pad(p[:, 0], (0, NPA - N))
    y1 = jnp.pad(p[:, 1], (0, NPA - N))
    x2 = jnp.pad(p[:, 2], (0, NPA - N))
    y2 = jnp.pad(p[:, 3], (0, NPA - N))
    sc = jnp.pad(p[:, 4], (0, NPA - N))
    g = jnp.pad(anchor_gt_inds.astype(jnp.int32), (0, NPA - N),
                constant_values=-1)
    gb = gt_bboxes.astype(jnp.float32)
    gx1 = jnp.pad(gb[:, 0], (0, GP - G))
    gy1 = jnp.pad(gb[:, 1], (0, GP - G))
    gx2 = jnp.pad(gb[:, 2], (0, GP - G))
    gy2 = jnp.pad(gb[:, 3], (0, GP - G))
    out = _nms_sc(x1, y1, x2, y2, sc, g, gx1, gy1, gx2, gy2)
    return jnp.stack([out[0], out[1]])


# speedup vs baseline: 26.1136x; 1.1037x over previous
import jax
import jax.numpy as jnp
from jax import lax
from jax.experimental import pallas as pl
from jax.experimental.pallas import tpu as pltpu
from jax.experimental.pallas import tpu_sc as plsc

N = 1000
NP = 1024
NPA = 1040
NCH = NP // 16
G = 50
GP = 80
GSTRIDE = 64
GT2P = G * GSTRIDE + 16
RECP = 80

NMS_THR = 0.5
EPS = 1e-6
LN2 = 0.6931471805599453
SQRT2 = 1.4142135623730951
NEG_INF = float("-inf")


def _vlog(x):
    bits = lax.bitcast_convert_type(x, jnp.int32)
    e = jnp.right_shift(bits, 23) & 0xFF
    ef = (e - 127).astype(jnp.float32)
    m = lax.bitcast_convert_type((bits & 0x007FFFFF) | 0x3F800000, jnp.float32)
    r = (m - 1.0) / (m + 1.0)
    r2 = r * r
    p = r2 * (2.0 / 7.0) + (2.0 / 5.0)
    p = p * r2 + (2.0 / 3.0)
    p = p * r2 + 2.0
    return ef * LN2 + r * p


def _bfly_sum(v, iota):
    for d in (1, 2, 4, 8):
        v = v + v[iota ^ d]
    return v


def _bfly_max(v, iota):
    for d in (1, 2, 4, 8):
        v = jnp.maximum(v, v[iota ^ d])
    return v


def _bfly_max_argmax(bm, bidx, iota):
    bmax = _bfly_max(bm, iota)
    sel = jnp.where(bm == bmax, bidx, -1)
    imax = _bfly_max(sel, iota)
    return bmax, imax


def _iou16(ax1, ay1, ax2, ay2, aarea, bx1, by1, bx2, by2, barea):
    ltx = jnp.maximum(ax1, bx1)
    lty = jnp.maximum(ay1, by1)
    rbx = jnp.minimum(ax2, bx2)
    rby = jnp.minimum(ay2, by2)
    w = jnp.maximum(rbx - ltx, 0.0)
    h = jnp.maximum(rby - lty, 0.0)
    inter = w * h
    union = jnp.maximum(aarea + barea - inter, EPS)
    return inter / union


def _nms_body(pt_h, g_h, gt_h, out_h,
              vx1, vy1, vx2, vy2, vsc, vg, varea, vact, vglane, vgtsel,
              vnsc, vgx1, vgy1, vgx2, vgy2, vgarea, vgt, vrec, vout, vtmp,
              sti, stf, sem):
    cid = lax.axis_index("c")
    sid = lax.axis_index("s")

    @pl.when((cid == 0) & (sid == 0))
    def _():
        iota = lax.iota(jnp.int32, 16)
        lane0 = iota == 0

        copies = [pltpu.async_copy(s, d, sem) for s, d in (
            (pt_h.at[pl.ds(0, NPA)], vx1),
            (pt_h.at[pl.ds(NPA, NPA)], vy1),
            (pt_h.at[pl.ds(2 * NPA, NPA)], vx2),
            (pt_h.at[pl.ds(3 * NPA, NPA)], vy2),
            (pt_h.at[pl.ds(4 * NPA, NPA)], vsc), (g_h, vg),
            (gt_h.at[pl.ds(0, GP)], vgx1),
            (gt_h.at[pl.ds(GP, GP)], vgy1),
            (gt_h.at[pl.ds(2 * GP, GP)], vgx2),
            (gt_h.at[pl.ds(3 * GP, GP)], vgy2))]
        for c in copies:
            c.wait()

        for c in range(GP // 16):
            sl = pl.ds(c * 16, 16)
            vgarea[sl] = (vgx2[sl] - vgx1[sl]) * (vgy2[sl] - vgy1[sl])
        for c in range(RECP // 16):
            vrec[pl.ds(c * 16, 16)] = jnp.full((16,), -1, jnp.int32)

        def gt_row(r, carry):
            rx1 = jnp.full((16,), vgx1[pl.ds(r, 16)][0], jnp.float32)
            ry1 = jnp.full((16,), vgy1[pl.ds(r, 16)][0], jnp.float32)
            rx2 = jnp.full((16,), vgx2[pl.ds(r, 16)][0], jnp.float32)
            ry2 = jnp.full((16,), vgy2[pl.ds(r, 16)][0], jnp.float32)
            rar = jnp.full((16,), vgarea[pl.ds(r, 16)][0], jnp.float32)
            for c in range(64 // 16):
                sl = pl.ds(c * 16, 16)
                v = _iou16(rx1, ry1, rx2, ry2, rar,
                           vgx1[sl], vgy1[sl], vgx2[sl], vgy2[sl],
                           vgarea[sl])
                vgt[pl.ds(r * GSTRIDE + c * 16, 16)] = v
            return carry

        lax.fori_loop(0, G, gt_row, 0)

        def init_body(c, carry):
            bm, bidx, cnt = carry
            sl = pl.ds(c * 16, 16)
            area = (vx2[sl] - vx1[sl]) * (vy2[sl] - vy1[sl])
            varea[sl] = area
            cg = vg[sl]
            cgc = jnp.maximum(cg, 0)
            vglane[sl] = cgc & 15
            vgtsel[sl] = jnp.right_shift(cgc, 4)
            vnsc[sl] = -vsc[sl]
            act = cg >= 0
            vact[sl] = jnp.where(act, 1, 0).astype(jnp.int32)
            masked = jnp.where(act, vsc[sl], NEG_INF)
            idxv = c * 16 + iota
            upd = masked >= bm
            bm = jnp.where(upd, masked, bm)
            bidx = jnp.where(upd, idxv, bidx)
            cnt = cnt + jnp.where(act, 1, 0).astype(jnp.int32)
            return bm, bidx, cnt

        bm0 = jnp.full((16,), NEG_INF, jnp.float32)
        bi0 = jnp.full((16,), -1, jnp.int32)
        cv0 = jnp.zeros((16,), jnp.int32)
        bm, bidx, cntv = lax.fori_loop(0, NCH, init_body, (bm0, bi0, cv0))
        bm, bidx = _bfly_max_argmax(bm, bidx, iota)
        sti[0] = bidx[0]
        sti[1] = _bfly_sum(cntv, iota)[0]
        sti[2] = 0
        sti[3] = 0
        stf[0] = bm[0]
        stf[1] = 0.0
        stf[2] = 0.0

        def outer(t, carry):
            act_cnt = sti[1]

            @pl.when(act_cnt > 0)
            def _():
                i = sti[0]
                top = stf[0]
                terminal = act_cnt == 1
                lane0_c = iota == 0
                av = vact[pl.ds(i, 16)]
                vact[pl.ds(i, 16)] = jnp.where(lane0_c, 0, av)
                sti[4] = vg[pl.ds(i, 16)][0]
                gi = sti[4]
                bx1s = vx1[pl.ds(i, 16)][0]
                by1s = vy1[pl.ds(i, 16)][0]
                bx2s = vx2[pl.ds(i, 16)][0]
                by2s = vy2[pl.ds(i, 16)][0]
                bars = varea[pl.ds(i, 16)][0]
                bx1 = jnp.full((16,), bx1s, jnp.float32)
                by1 = jnp.full((16,), by1s, jnp.float32)
                bx2 = jnp.full((16,), bx2s, jnp.float32)
                by2 = jnp.full((16,), by2s, jnp.float32)
                bar = jnp.full((16,), bars, jnp.float32)

                lane0_o = iota == 0
                rv = vrec[pl.ds(gi, 16)]
                sti[5] = rv[0]
                mi = sti[5]
                newr = jnp.where(mi < 0, i, mi)
                vrec[pl.ds(gi, 16)] = jnp.where(lane0_o, newr, rv)
                has_pull = mi >= 0

                sti[6] = jnp.maximum(mi, 0)
                mic = sti[6]
                mx1 = jnp.full((16,), vx1[pl.ds(mic, 16)][0], jnp.float32)
                my1 = jnp.full((16,), vy1[pl.ds(mic, 16)][0], jnp.float32)
                mx2 = jnp.full((16,), vx2[pl.ds(mic, 16)][0], jnp.float32)
                my2 = jnp.full((16,), vy2[pl.ds(mic, 16)][0], jnp.float32)
                mar = jnp.full((16,), varea[pl.ds(mic, 16)][0], jnp.float32)
                iou_mi = _iou16(bx1, by1, bx2, by2, bar,
                                mx1, my1, mx2, my2, mar)
                ms = jnp.maximum(iou_mi, EPS)
                vtmp[...] = -_vlog(ms + (1.0 - NMS_THR)) * top
                pull = vtmp[...][0]
                add_pull = jnp.where(has_pull, 1, 0) * jnp.where(terminal, 0, 1)
                stf[1] = stf[1] + jnp.where(add_pull != 0, pull, 0.0)
                sti[2] = sti[2] + jnp.where(has_pull, 1, 0)

                gb = gi * GSTRIDE
                gr0 = jnp.maximum(vgt[pl.ds(gb, 16)], NMS_THR)
                gr1 = jnp.maximum(vgt[pl.ds(gb + 16, 16)], NMS_THR)
                gr2 = jnp.maximum(vgt[pl.ds(gb + 32, 16)], NMS_THR)
                gr3 = jnp.maximum(vgt[pl.ds(gb + 48, 16)], NMS_THR)

                def sweep(c, carry):
                    sbm, sbidx, scnt, spush, skv = carry
                    sl = pl.ds(c * 16, 16)
                    csc = vsc[sl]
                    cg = vg[sl]
                    acti = vact[sl]
                    idxv = c * 16 + iota
                    row = _iou16(bx1, by1, bx2, by2, bar,
                                 vx1[sl], vy1[sl], vx2[sl], vy2[sl],
                                 varea[sl])
                    resti = acti
                    nacti = resti & jnp.where(row > NMS_THR, 0, 1
                                              ).astype(jnp.int32)
                    vact[sl] = nacti
                    lanei = vglane[sl]
                    tsel = vgtsel[sl]
                    gtv = jnp.where(
                        tsel == 0, gr0[lanei],
                        jnp.where(tsel == 1, gr1[lanei],
                                  jnp.where(tsel == 2, gr2[lanei],
                                            gr3[lanei])))
                    candi = (resti
                             & jnp.where(cg != gi, 1, 0).astype(jnp.int32)
                             & jnp.where(row > gtv, 1, 0).astype(jnp.int32))
                    cand = candi != 0
                    lg = _vlog(1.0 - row)
                    spush = spush + jnp.where(cand, lg * vnsc[sl], 0.0)
                    skv = skv + candi
                    nact = nacti != 0
                    masked = jnp.where(nact, csc, NEG_INF)
                    upd = masked >= sbm
                    sbm = jnp.where(upd, masked, sbm)
                    sbidx = jnp.where(upd, idxv, sbidx)
                    scnt = scnt + nacti
                    return sbm, sbidx, scnt, spush, skv

                z_i = jnp.zeros((16,), jnp.int32)
                z_f = jnp.zeros((16,), jnp.float32)
                sbm, sbidx, scnt, spush, skv = lax.fori_loop(
                    0, NCH, sweep,
                    (jnp.full((16,), NEG_INF, jnp.float32),
                     jnp.full((16,), -1, jnp.int32), z_i, z_f, z_i))

                k = _bfly_sum(skv, iota)[0]
                s = _bfly_sum(spush, iota)[0]
                add_push = jnp.where(k > 0, 1, 0) * jnp.where(terminal, 0, 1)
                kf = jnp.maximum(k, 1).astype(jnp.float32)
                vtmp[...] = jnp.full((16,), s, jnp.float32) / jnp.full(
                    (16,), kf, jnp.float32)
                stf[2] = stf[2] + jnp.where(add_push != 0, vtmp[...][0], 0.0)
                sti[3] = sti[3] + k
                sbm, sbidx = _bfly_max_argmax(sbm, sbidx, iota)
                sti[0] = sbidx[0]
                stf[0] = sbm[0]
                sti[1] = _bfly_sum(scnt, iota)[0]

            return carry

        def outer_blk(b, carry):
            @pl.when(sti[1] > 0)
            def _():
                lax.fori_loop(0, 16, outer, 0)
            return carry

        lax.fori_loop(0, NP // 16, outer_blk, 0)

        pull_loss = jnp.full((16,), stf[1], jnp.float32) / jnp.full(
            (16,), sti[2].astype(jnp.float32) + EPS, jnp.float32)
        push_loss = jnp.full((16,), stf[2], jnp.float32) / jnp.full(
            (16,), sti[3].astype(jnp.float32) + EPS, jnp.float32)
        res = jnp.where(lane0, push_loss,
                        jnp.where(iota == 1, pull_loss, 0.0))
        vout[...] = res
        pltpu.sync_copy(vout, out_h)


@jax.jit
def _nms_sc(pt, g, gt):
    mesh = plsc.VectorSubcoreMesh(core_axis_name="c", subcore_axis_name="s")
    f = pl.kernel(
        _nms_body,
        out_type=jax.ShapeDtypeStruct((16,), jnp.float32),
        mesh=mesh,
        scratch_types=[
            pltpu.VMEM((NPA,), jnp.float32),
            pltpu.VMEM((NPA,), jnp.float32),
            pltpu.VMEM((NPA,), jnp.float32),
            pltpu.VMEM((NPA,), jnp.float32),
            pltpu.VMEM((NPA,), jnp.float32),
            pltpu.VMEM((NPA,), jnp.int32),
            pltpu.VMEM((NPA,), jnp.float32),
            pltpu.VMEM((NPA,), jnp.int32),
            pltpu.VMEM((NP,), jnp.int32),
            pltpu.VMEM((NP,), jnp.int32),
            pltpu.VMEM((NP,), jnp.float32),
            pltpu.VMEM((GP,), jnp.float32),
            pltpu.VMEM((GP,), jnp.float32),
            pltpu.VMEM((GP,), jnp.float32),
            pltpu.VMEM((GP,), jnp.float32),
            pltpu.VMEM((GP,), jnp.float32),
            pltpu.VMEM((GT2P,), jnp.float32),
            pltpu.VMEM((RECP,), jnp.int32),
            pltpu.VMEM((16,), jnp.float32),
            pltpu.VMEM((16,), jnp.float32),
            pltpu.SMEM((8,), jnp.int32),
            pltpu.SMEM((8,), jnp.float32),
            pltpu.SemaphoreType.DMA,
        ],
    )
    return f(pt, g, gt)


def kernel(gt_inds, anchor_gt_inds, gt_bboxes, proposal_list):
    del gt_inds
    pt = jnp.pad(proposal_list.astype(jnp.float32).T,
                 ((0, 0), (0, NPA - N))).reshape(-1)
    g = jnp.pad(anchor_gt_inds.astype(jnp.int32), (0, NPA - N),
                constant_values=-1)
    gt = jnp.pad(gt_bboxes.astype(jnp.float32).T,
                 ((0, 0), (0, GP - G))).reshape(-1)
    out = _nms_sc(pt, g, gt)
    return jnp.stack([out[0], out[1]])


# speedup vs baseline: 37.4307x; 1.4334x over previous
import jax
import jax.numpy as jnp
from jax import lax
from jax.experimental import pallas as pl
from jax.experimental.pallas import tpu as pltpu
from jax.experimental.pallas import tpu_sc as plsc

N = 1000
NP = 1024
NPA = 1040
NCH = NP // 16
G = 50
GP = 80
GSTRIDE = 64
GT2P = G * GSTRIDE + 16
RECP = 80

NMS_THR = 0.5
EPS = 1e-6
LN2 = 0.6931471805599453
NEG_INF = float("-inf")


def _vlog(x):
    bits = lax.bitcast_convert_type(x, jnp.int32)
    e = jnp.right_shift(bits, 23) & 0xFF
    ef = (e - 127).astype(jnp.float32)
    m = lax.bitcast_convert_type((bits & 0x007FFFFF) | 0x3F800000, jnp.float32)
    r = (m - 1.0) / (m + 1.0)
    r2 = r * r
    p = r2 * (2.0 / 7.0) + (2.0 / 5.0)
    p = p * r2 + (2.0 / 3.0)
    p = p * r2 + 2.0
    return ef * LN2 + r * p


def _bfly_sum(v, iota):
    for d in (1, 2, 4, 8):
        v = v + v[iota ^ d]
    return v


def _bfly_max(v, iota):
    for d in (1, 2, 4, 8):
        v = jnp.maximum(v, v[iota ^ d])
    return v


def _bfly_max_argmax(bm, bidx, iota):
    bmax = _bfly_max(bm, iota)
    sel = jnp.where(bm == bmax, bidx, -1)
    imax = _bfly_max(sel, iota)
    return bmax, imax


def _iou16(ax1, ay1, ax2, ay2, aarea, bx1, by1, bx2, by2, barea):
    ltx = jnp.maximum(ax1, bx1)
    lty = jnp.maximum(ay1, by1)
    rbx = jnp.minimum(ax2, bx2)
    rby = jnp.minimum(ay2, by2)
    w = jnp.maximum(rbx - ltx, 0.0)
    h = jnp.maximum(rby - lty, 0.0)
    inter = w * h
    union = jnp.maximum(aarea + barea - inter, EPS)
    return inter / union


def _nms_body(pt_h, g_h, gt_h, out_h,
              vx1, vy1, vx2, vy2, vsc, vg, varea, vact, vglane, vgtsel,
              vnsc, vgx1, vgy1, vgx2, vgy2, vgarea, vgt, vrec, vout, vtmp,
              vshl, vloc, vbc, shp, shb, sti, stf, sem):
    cid = lax.axis_index("c")
    sid = lax.axis_index("s")

    @pl.when(cid == 0)
    def _():
        w = sid
        iota = lax.iota(jnp.int32, 16)
        lane0 = iota == 0

        copies = [pltpu.async_copy(s, d, sem) for s, d in (
            (pt_h.at[pl.ds(0, NPA)], vx1),
            (pt_h.at[pl.ds(NPA, NPA)], vy1),
            (pt_h.at[pl.ds(2 * NPA, NPA)], vx2),
            (pt_h.at[pl.ds(3 * NPA, NPA)], vy2),
            (pt_h.at[pl.ds(4 * NPA, NPA)], vsc), (g_h, vg),
            (gt_h.at[pl.ds(0, GP)], vgx1),
            (gt_h.at[pl.ds(GP, GP)], vgy1),
            (gt_h.at[pl.ds(2 * GP, GP)], vgx2),
            (gt_h.at[pl.ds(3 * GP, GP)], vgy2))]
        for c in copies:
            c.wait()

        for c in range(GP // 16):
            sl = pl.ds(c * 16, 16)
            vgarea[sl] = (vgx2[sl] - vgx1[sl]) * (vgy2[sl] - vgy1[sl])
        for c in range(RECP // 16):
            vrec[pl.ds(c * 16, 16)] = jnp.full((16,), -1, jnp.int32)

        def gt_row(r, carry):
            rx1 = jnp.full((16,), vgx1[pl.ds(r, 16)][0], jnp.float32)
            ry1 = jnp.full((16,), vgy1[pl.ds(r, 16)][0], jnp.float32)
            rx2 = jnp.full((16,), vgx2[pl.ds(r, 16)][0], jnp.float32)
            ry2 = jnp.full((16,), vgy2[pl.ds(r, 16)][0], jnp.float32)
            rar = jnp.full((16,), vgarea[pl.ds(r, 16)][0], jnp.float32)
            for c in range(64 // 16):
                sl = pl.ds(c * 16, 16)
                v = _iou16(rx1, ry1, rx2, ry2, rar,
                           vgx1[sl], vgy1[sl], vgx2[sl], vgy2[sl],
                           vgarea[sl])
                vgt[pl.ds(r * GSTRIDE + c * 16, 16)] = v
            return carry

        lax.fori_loop(0, G, gt_row, 0)

        def init_body(c, carry):
            bm, bidx, cnt = carry
            sl = pl.ds(c * 16, 16)
            area = (vx2[sl] - vx1[sl]) * (vy2[sl] - vy1[sl])
            varea[sl] = area
            cg = vg[sl]
            cgc = jnp.maximum(cg, 0)
            vglane[sl] = cgc & 15
            vgtsel[sl] = jnp.right_shift(cgc, 4)
            vnsc[sl] = -vsc[sl]
            act = cg >= 0
            vact[sl] = jnp.where(act, 1, 0).astype(jnp.int32)
            masked = jnp.where(act, vsc[sl], NEG_INF)
            idxv = c * 16 + iota
            upd = masked >= bm
            bm = jnp.where(upd, masked, bm)
            bidx = jnp.where(upd, idxv, bidx)
            cnt = cnt + jnp.where(act, 1, 0).astype(jnp.int32)
            return bm, bidx, cnt

        bm0 = jnp.full((16,), NEG_INF, jnp.float32)
        bi0 = jnp.full((16,), -1, jnp.int32)
        cv0 = jnp.zeros((16,), jnp.int32)
        bm, bidx, cntv = lax.fori_loop(0, NCH, init_body, (bm0, bi0, cv0))
        bm, bidx = _bfly_max_argmax(bm, bidx, iota)
        sti[7] = 0
        sti[0] = bidx[0]
        sti[1] = _bfly_sum(cntv, iota)[0]
        sti[2] = 0
        sti[3] = 0
        stf[0] = bm[0]
        stf[1] = 0.0
        stf[2] = 0.0

        def outer(t, carry):
            act_cnt = sti[1]

            @pl.when(act_cnt > 0)
            def _():
                i = sti[0]
                top = stf[0]
                terminal = act_cnt == 1
                lane0_c = iota == 0
                av = vact[pl.ds(i, 16)]
                vact[pl.ds(i, 16)] = jnp.where(lane0_c, 0, av)
                sti[4] = vg[pl.ds(i, 16)][0]
                gi = sti[4]
                bx1s = vx1[pl.ds(i, 16)][0]
                by1s = vy1[pl.ds(i, 16)][0]
                bx2s = vx2[pl.ds(i, 16)][0]
                by2s = vy2[pl.ds(i, 16)][0]
                bars = varea[pl.ds(i, 16)][0]
                bx1 = jnp.full((16,), bx1s, jnp.float32)
                by1 = jnp.full((16,), by1s, jnp.float32)
                bx2 = jnp.full((16,), bx2s, jnp.float32)
                by2 = jnp.full((16,), by2s, jnp.float32)
                bar = jnp.full((16,), bars, jnp.float32)

                @pl.when(w == 15)
                def _():
                    lane0_o = iota == 0
                    rv = vrec[pl.ds(gi, 16)]
                    sti[5] = rv[0]
                    mi = sti[5]
                    newr = jnp.where(mi < 0, i, mi)
                    vrec[pl.ds(gi, 16)] = jnp.where(lane0_o, newr, rv)

                    @pl.when(mi >= 0)
                    def _():
                        mic = sti[5]
                        mx1 = jnp.full((16,), vx1[pl.ds(mic, 16)][0],
                                       jnp.float32)
                        my1 = jnp.full((16,), vy1[pl.ds(mic, 16)][0],
                                       jnp.float32)
                        mx2 = jnp.full((16,), vx2[pl.ds(mic, 16)][0],
                                       jnp.float32)
                        my2 = jnp.full((16,), vy2[pl.ds(mic, 16)][0],
                                       jnp.float32)
                        mar = jnp.full((16,), varea[pl.ds(mic, 16)][0],
                                       jnp.float32)
                        iou_mi = _iou16(bx1, by1, bx2, by2, bar,
                                        mx1, my1, mx2, my2, mar)
                        ms = jnp.maximum(iou_mi, EPS)
                        vtmp[...] = -_vlog(ms + (1.0 - NMS_THR)) * top
                        pull = vtmp[...][0]
                        stf[1] = stf[1] + jnp.where(terminal, 0.0, pull)
                        sti[2] = sti[2] + 1

                gb = gi * GSTRIDE
                gr0 = jnp.maximum(vgt[pl.ds(gb, 16)], NMS_THR)
                gr1 = jnp.maximum(vgt[pl.ds(gb + 16, 16)], NMS_THR)
                gr2 = jnp.maximum(vgt[pl.ds(gb + 32, 16)], NMS_THR)
                gr3 = jnp.maximum(vgt[pl.ds(gb + 48, 16)], NMS_THR)

                def sweep(c, carry):
                    sbm, sbidx, scnt, spush, skv = carry
                    gc = w * 4 + c
                    sl = pl.ds(gc * 16, 16)
                    csc = vsc[sl]
                    cg = vg[sl]
                    acti = vact[sl]
                    idxv = gc * 16 + iota
                    row = _iou16(bx1, by1, bx2, by2, bar,
                                 vx1[sl], vy1[sl], vx2[sl], vy2[sl],
                                 varea[sl])
                    resti = acti
                    nacti = resti & jnp.where(row > NMS_THR, 0, 1
                                              ).astype(jnp.int32)
                    vact[sl] = nacti
                    lanei = vglane[sl]
                    tsel = vgtsel[sl]
                    gtv = jnp.where(
                        tsel == 0, gr0[lanei],
                        jnp.where(tsel == 1, gr1[lanei],
                                  jnp.where(tsel == 2, gr2[lanei],
                                            gr3[lanei])))
                    candi = (resti
                             & jnp.where(cg != gi, 1, 0).astype(jnp.int32)
                             & jnp.where(row > gtv, 1, 0).astype(jnp.int32))
                    cand = candi != 0
                    lg = _vlog(1.0 - row)
                    spush = spush + jnp.where(cand, lg * vnsc[sl], 0.0)
                    skv = skv + candi
                    nact = nacti != 0
                    masked = jnp.where(nact, csc, NEG_INF)
                    upd = masked >= sbm
                    sbm = jnp.where(upd, masked, sbm)
                    sbidx = jnp.where(upd, idxv, sbidx)
                    scnt = scnt + nacti
                    return sbm, sbidx, scnt, spush, skv

                z_i = jnp.zeros((16,), jnp.int32)
                z_f = jnp.zeros((16,), jnp.float32)
                sbm, sbidx, scnt, spush, skv = lax.fori_loop(
                    0, 4, sweep,
                    (jnp.full((16,), NEG_INF, jnp.float32),
                     jnp.full((16,), -1, jnp.int32), z_i, z_f, z_i))

                sbm, sbidx = _bfly_max_argmax(sbm, sbidx, iota)
                cpv = _bfly_sum(scnt, iota).astype(jnp.float32)
                spv0 = _bfly_sum(spush, iota)
                kpv = _bfly_sum(skv, iota).astype(jnp.float32)
                mytile = iota == w
                vloc[pl.ds(0, 16)] = jnp.where(mytile, sbm, NEG_INF)
                vloc[pl.ds(16, 16)] = jnp.where(
                    mytile, sbidx.astype(jnp.float32), -1.0)
                vloc[pl.ds(32, 16)] = jnp.where(mytile, cpv, 0.0)
                vloc[pl.ds(48, 16)] = jnp.where(mytile, spv0, 0.0)
                vloc[pl.ds(64, 16)] = jnp.where(mytile, kpv, 0.0)
                pb = sti[7] * 1280
                pltpu.sync_copy(vloc, shp.at[pl.ds(pb + w * 80, 80)])
                plsc.subcore_barrier()

                pltpu.sync_copy(shp.at[pl.ds(pb, 1280)], vshl)
                sti[7] = 1 - sti[7]
                bmv = vshl[pl.ds(0, 16)]
                idv = vshl[pl.ds(16, 16)]
                cnv = vshl[pl.ds(32, 16)]
                spv = vshl[pl.ds(48, 16)]
                skvv = vshl[pl.ds(64, 16)]
                for ww in range(1, 16):
                    o = ww * 80
                    bmv = jnp.maximum(bmv, vshl[pl.ds(o, 16)])
                    idv = jnp.maximum(idv, vshl[pl.ds(o + 16, 16)])
                    cnv = cnv + vshl[pl.ds(o + 32, 16)]
                    spv = spv + vshl[pl.ds(o + 48, 16)]
                    skvv = skvv + vshl[pl.ds(o + 64, 16)]
                bmax, imax = _bfly_max_argmax(bmv, idv, iota)
                cnt_all = _bfly_sum(cnv, iota)
                vtmp[...] = jnp.where(iota == 0, imax,
                                      jnp.where(iota == 1, bmax, cnt_all))
                bcv = vtmp[...]
                sti[0] = bcv[0].astype(jnp.int32)
                stf[0] = bcv[1]
                sti[1] = bcv[2].astype(jnp.int32)

                @pl.when(w == 0)
                def _():
                    k = _bfly_sum(skvv, iota)[0]
                    s = _bfly_sum(spv, iota)[0]
                    add_push = jnp.where(k > 0.5, 1, 0) * jnp.where(
                        terminal, 0, 1)
                    kf = jnp.maximum(k, 1.0)
                    vtmp[...] = jnp.full((16,), s, jnp.float32) / jnp.full(
                        (16,), kf, jnp.float32)
                    stf[2] = stf[2] + jnp.where(
                        add_push != 0, vtmp[...][0], 0.0)
                    sti[3] = sti[3] + k.astype(jnp.int32)

            return carry

        def outer_blk(b, carry):
            @pl.when(sti[1] > 0)
            def _():
                lax.fori_loop(0, 16, outer, 0)
            return carry

        lax.fori_loop(0, NP // 16, outer_blk, 0)

        @pl.when(w == 15)
        def _():
            vtmp[...] = jnp.where(iota == 0, jnp.full((16,), stf[1]),
                                  jnp.full((16,), sti[2].astype(jnp.float32)))
            pltpu.sync_copy(vtmp, shb)

        plsc.subcore_barrier()

        @pl.when(w == 0)
        def _():
            pltpu.sync_copy(shb, vbc)
            pv = vbc[...]
            pull_loss = jnp.full((16,), pv[0], jnp.float32) / jnp.full(
                (16,), pv[1] + EPS, jnp.float32)
            push_loss = jnp.full((16,), stf[2], jnp.float32) / jnp.full(
                (16,), sti[3].astype(jnp.float32) + EPS, jnp.float32)
            res = jnp.where(lane0, push_loss,
                            jnp.where(iota == 1, pull_loss, 0.0))
            vout[...] = res
            pltpu.sync_copy(vout, out_h)


@jax.jit
def _nms_sc(pt, g, gt):
    mesh = plsc.VectorSubcoreMesh(core_axis_name="c", subcore_axis_name="s")
    f = pl.kernel(
        _nms_body,
        out_type=jax.ShapeDtypeStruct((16,), jnp.float32),
        mesh=mesh,
        scratch_types=[
            pltpu.VMEM((NPA,), jnp.float32),
            pltpu.VMEM((NPA,), jnp.float32),
            pltpu.VMEM((NPA,), jnp.float32),
            pltpu.VMEM((NPA,), jnp.float32),
            pltpu.VMEM((NPA,), jnp.float32),
            pltpu.VMEM((NPA,), jnp.int32),
            pltpu.VMEM((NPA,), jnp.float32),
            pltpu.VMEM((NPA,), jnp.int32),
            pltpu.VMEM((NP,), jnp.int32),
            pltpu.VMEM((NP,), jnp.int32),
            pltpu.VMEM((NP,), jnp.float32),
            pltpu.VMEM((GP,), jnp.float32),
            pltpu.VMEM((GP,), jnp.float32),
            pltpu.VMEM((GP,), jnp.float32),
            pltpu.VMEM((GP,), jnp.float32),
            pltpu.VMEM((GP,), jnp.float32),
            pltpu.VMEM((GT2P,), jnp.float32),
            pltpu.VMEM((RECP,), jnp.int32),
            pltpu.VMEM((16,), jnp.float32),
            pltpu.VMEM((16,), jnp.float32),
            pltpu.VMEM((1280,), jnp.float32),
            pltpu.VMEM((80,), jnp.float32),
            pltpu.VMEM((16,), jnp.float32),
            pltpu.VMEM_SHARED((2560,), jnp.float32),
            pltpu.VMEM_SHARED((16,), jnp.float32),
            pltpu.SMEM((8,), jnp.int32),
            pltpu.SMEM((8,), jnp.float32),
            pltpu.SemaphoreType.DMA,
        ],
    )
    return f(pt, g, gt)


def kernel(gt_inds, anchor_gt_inds, gt_bboxes, proposal_list):
    del gt_inds
    pt = jnp.pad(proposal_list.astype(jnp.float32).T,
                 ((0, 0), (0, NPA - N))).reshape(-1)
    g = jnp.pad(anchor_gt_inds.astype(jnp.int32), (0, NPA - N),
                constant_values=-1)
    gt = jnp.pad(gt_bboxes.astype(jnp.float32).T,
                 ((0, 0), (0, GP - G))).reshape(-1)
    out = _nms_sc(pt, g, gt)
    return jnp.stack([out[0], out[1]])
